# baseline (speedup 1.0000x reference)
import functools

import jax
import jax.numpy as jnp
from jax import lax
from jax.experimental import pallas as pl
from jax.experimental.pallas import tpu as pltpu
from jax.experimental.pallas import tpu_sc as plsc

NC = 2
NS = 16
LANES = 16
CH = 256
NBUF = 6


@functools.lru_cache(maxsize=None)
def _sc_scatter_kernel(n, d, e):
    dh = d // NC
    assert d % NC == 0 and dh % LANES == 0
    epw = e // NS
    assert e % NS == 0 and epw % 8 == 0
    nch = epw // CH
    tail = epw - nch * CH
    assert nch % NBUF == 0 and nch >= NBUF
    wb = (n // NS) & ~7
    rem = n - NS * wb
    zb = next(z for z in range(min(CH, wb), 0, -1)
              if wb % z == 0 and z % 8 == 0)
    assert rem % 8 == 0 and rem <= wb

    def body(x_hbm, edge_hbm, sum_hbm, deg_hbm, *refs):
        src_v = refs[0:NBUF]
        dst_v = refs[NBUF:2 * NBUF]
        rows_v = refs[2 * NBUF:3 * NBUF]
        src_t, dst_t, rows_t, ones_v, zdeg_v, acc_sh, accd_sh = \
            refs[3 * NBUF:3 * NBUF + 7]
        semi = refs[3 * NBUF + 7:3 * NBUF + 7 + NBUF]
        semg = refs[3 * NBUF + 7 + NBUF:3 * NBUF + 7 + 2 * NBUF]
        sems = refs[3 * NBUF + 7 + 2 * NBUF:3 * NBUF + 7 + 3 * NBUF]
        semd = refs[3 * NBUF + 7 + 3 * NBUF:3 * NBUF + 7 + 4 * NBUF]
        semt = refs[3 * NBUF + 7 + 4 * NBUF]

        c = lax.axis_index("c")
        s = lax.axis_index("s")
        zero16 = jnp.zeros((LANES,), jnp.float32)
        zero32b = jnp.zeros((2 * LANES,), jnp.bfloat16)
        e0 = jnp.where(lax.iota(jnp.int32, LANES) == 0,
                       jnp.float32(1.0), jnp.float32(0.0))
        row_off = c * n

        def idx_issue(b, base):
            pltpu.async_copy(edge_hbm.at[pl.ds(base, CH)], src_v[b], semi[b])
            pltpu.async_copy(edge_hbm.at[pl.ds(e + base, CH)], dst_v[b],
                             semi[b])

        def idx_wait(b):
            pltpu.make_async_copy(edge_hbm.at[pl.ds(0, CH)], src_v[b],
                                  semi[b]).wait()
            pltpu.make_async_copy(edge_hbm.at[pl.ds(0, CH)], dst_v[b],
                                  semi[b]).wait()

        def gather_issue(b):
            for j in range(CH // LANES):
                sl = pl.ds(j * LANES, LANES)
                src_v[b][sl] = src_v[b][sl] + row_off
            pltpu.async_copy(x_hbm.at[src_v[b]], rows_v[b], semg[b])

        def gather_wait(b):
            pltpu.make_async_copy(x_hbm.at[src_v[b]], rows_v[b],
                                  semg[b]).wait()

        def scat_issue(b):
            pltpu.async_copy(rows_v[b], acc_sh.at[dst_v[b]], sems[b],
                             add=True)
            @pl.when(c == b % 2)
            def _():
                pltpu.async_copy(ones_v, accd_sh.at[dst_v[b]], semd[b],
                                 add=True)

        def scat_wait(b):
            pltpu.make_async_copy(rows_v[b], acc_sh.at[dst_v[b]],
                                  sems[b]).wait()

            @pl.when(c == b % 2)
            def _():
                pltpu.make_async_copy(ones_v, accd_sh.at[dst_v[b]],
                                      semd[b]).wait()

        def zrow(i, carry):
            for j in range(dh // (2 * LANES)):
                rows_v[0][i, pl.ds(j * 2 * LANES, 2 * LANES)] = zero32b
            ones_v[i, :] = e0
            return carry
        lax.fori_loop(0, CH, zrow, 0)

        def zdeg(i, carry):
            zdeg_v[i, :] = zero16
            return carry
        lax.fori_loop(0, wb, zdeg, 0)

        base_row = s * wb
        for j in range(wb // zb):
            pltpu.sync_copy(rows_v[0].at[pl.ds(0, zb)],
                            acc_sh.at[pl.ds(base_row + j * zb, zb)])
        pltpu.sync_copy(zdeg_v, accd_sh.at[pl.ds(base_row, wb)])
        if rem:
            @pl.when(s == 0)
            def _():
                pltpu.sync_copy(rows_v[0].at[pl.ds(0, rem)],
                                acc_sh.at[pl.ds(NS * wb, rem)])
                pltpu.sync_copy(zdeg_v.at[pl.ds(0, rem)],
                                accd_sh.at[pl.ds(NS * wb, rem)])
        plsc.subcore_barrier()

        ebase = s * epw

        pltpu.sync_copy(edge_hbm.at[pl.ds(ebase, CH)], src_v[0])
        pltpu.sync_copy(edge_hbm.at[pl.ds(e + ebase, CH)], dst_v[0])
        gather_issue(0)
        idx_issue(1, ebase + CH)
        idx_wait(1)
        gather_issue(1)
        idx_issue(2, ebase + 2 * CH)

        def sext(g, carry):
            for b in range(NBUF):
                i = g * NBUF + b
                b2 = (b + 2) % NBUF
                b3 = (b + 3) % NBUF

                @pl.when(i >= 3)
                def _():
                    scat_wait(b3)

                @pl.when(i + 3 < nch)
                def _():
                    idx_issue(b3, ebase + (i + 3) * CH)

                @pl.when(i + 2 < nch)
                def _():
                    idx_wait(b2)
                    gather_issue(b2)

                gather_wait(b)
                scat_issue(b)
            return carry
        lax.fori_loop(0, nch // NBUF, sext, 0)
        for k in (nch - 3, nch - 2, nch - 1):
            scat_wait(k % NBUF)

        if tail:
            bt = ebase + nch * CH
            pltpu.sync_copy(edge_hbm.at[pl.ds(bt, tail)], src_t)
            pltpu.sync_copy(edge_hbm.at[pl.ds(e + bt, tail)], dst_t)
            for j in range(tail // LANES):
                sl = pl.ds(j * LANES, LANES)
                src_t[sl] = src_t[sl] + row_off
            pltpu.async_copy(x_hbm.at[src_t], rows_t, semt).wait()
            pltpu.sync_copy(rows_t, acc_sh.at[dst_t], add=True)

            @pl.when(c == 1)
            def _():
                pltpu.sync_copy(ones_v.at[pl.ds(0, tail)],
                                accd_sh.at[dst_t], add=True)
        plsc.subcore_barrier()

        out_base = c * n + base_row
        pltpu.sync_copy(acc_sh.at[pl.ds(base_row, wb)],
                        sum_hbm.at[pl.ds(out_base, wb)])
        pltpu.sync_copy(accd_sh.at[pl.ds(base_row, wb)],
                        deg_hbm.at[pl.ds(out_base, wb)])
        if rem:
            @pl.when(s == 0)
            def _():
                pltpu.sync_copy(acc_sh.at[pl.ds(NS * wb, rem)],
                                sum_hbm.at[pl.ds(c * n + NS * wb, rem)])
                pltpu.sync_copy(accd_sh.at[pl.ds(NS * wb, rem)],
                                deg_hbm.at[pl.ds(c * n + NS * wb, rem)])

    scratch = (
        [pltpu.VMEM((CH,), jnp.int32) for _ in range(NBUF)]
        + [pltpu.VMEM((CH,), jnp.int32) for _ in range(NBUF)]
        + [pltpu.VMEM((CH, dh), jnp.bfloat16) for _ in range(NBUF)]
        + [
            pltpu.VMEM((max(tail, 8),), jnp.int32),
            pltpu.VMEM((max(tail, 8),), jnp.int32),
            pltpu.VMEM((max(tail, 8), dh), jnp.bfloat16),
            pltpu.VMEM((CH, LANES), jnp.float32),
            pltpu.VMEM((wb, LANES), jnp.float32),
            pltpu.MemorySpace.VMEM_SHARED((n, dh), jnp.bfloat16),
            pltpu.MemorySpace.VMEM_SHARED((n, LANES), jnp.float32),
        ]
        + [pltpu.SemaphoreType.DMA for _ in range(4 * NBUF + 1)]
    )

    return pl.kernel(
        body,
        out_type=(jax.ShapeDtypeStruct((NC * n, dh), jnp.bfloat16),
                  jax.ShapeDtypeStruct((NC * n, LANES), jnp.float32)),
        mesh=plsc.VectorSubcoreMesh(core_axis_name="c", subcore_axis_name="s"),
        compiler_params=pltpu.CompilerParams(use_tc_tiling_on_sc=False),
        scratch_types=scratch,
    )


def _tc_self_body(x_ref, w_ref, b_ref, o_ref):
    o_ref[...] = (
        jnp.dot(x_ref[...], w_ref[0], preferred_element_type=jnp.float32)
        + b_ref[...])


def _tc_body(o1_ref, s_ref, d_ref, w_ref, o_ref):
    ssum = jnp.concatenate([s_ref[0], s_ref[1]],
                           axis=1).astype(jnp.float32)
    deg = jnp.sum(d_ref[0] + d_ref[1], axis=1, keepdims=True)
    h = ssum / jnp.maximum(deg, 1.0)
    o_ref[...] = (
        o1_ref[...]
        + jnp.dot(h, w_ref[1], preferred_element_type=jnp.float32))


def kernel(x, edge_index, W_neigh, b_neigh):
    n, d = x.shape
    e = edge_index.shape[1]
    dout = W_neigh.shape[1]
    dh = d // NC

    xs = x.reshape(n, NC, dh).swapaxes(0, 1).reshape(NC * n, dh)
    xs = xs.astype(jnp.bfloat16)
    sums, degs = _sc_scatter_kernel(n, d, e)(xs, edge_index.reshape(-1))
    sums = sums.reshape(NC, n, dh)
    degs = degs.reshape(NC, n, LANES)
    w = W_neigh.reshape(2, d, dout)
    b = b_neigh.reshape(1, dout)

    rb = n
    out1 = pl.pallas_call(
        _tc_self_body,
        grid=(n // rb,),
        in_specs=[
            pl.BlockSpec((rb, d), lambda i: (i, 0)),
            pl.BlockSpec((2, d, dout), lambda i: (0, 0, 0)),
            pl.BlockSpec((1, dout), lambda i: (0, 0)),
        ],
        out_specs=pl.BlockSpec((rb, dout), lambda i: (i, 0)),
        out_shape=jax.ShapeDtypeStruct((n, dout), jnp.float32),
    )(x, w, b)
    out = pl.pallas_call(
        _tc_body,
        grid=(n // rb,),
        in_specs=[
            pl.BlockSpec((rb, dout), lambda i: (i, 0)),
            pl.BlockSpec((NC, rb, dh), lambda i: (0, i, 0)),
            pl.BlockSpec((NC, rb, LANES), lambda i: (0, i, 0)),
            pl.BlockSpec((2, d, dout), lambda i: (0, 0, 0)),
        ],
        out_specs=pl.BlockSpec((rb, dout), lambda i: (i, 0)),
        out_shape=jax.ShapeDtypeStruct((n, dout), jnp.float32),
    )(out1, sums, degs, w)
    return out


# speedup vs baseline: 1.0023x; 1.0023x over previous
import functools

import jax
import jax.numpy as jnp
from jax import lax
from jax.experimental import pallas as pl
from jax.experimental.pallas import tpu as pltpu
from jax.experimental.pallas import tpu_sc as plsc

NC = 2
NS = 16
LANES = 16
CH = 256
NBUF = 6


@functools.lru_cache(maxsize=None)
def _sc_scatter_kernel(n, d, e):
    dh = d // NC
    assert d % NC == 0 and dh % LANES == 0
    epw = e // NS
    assert e % NS == 0 and epw % 8 == 0
    nch = epw // CH
    tail = epw - nch * CH
    assert nch % NBUF == 0 and nch >= NBUF
    wb = (n // NS) & ~7
    rem = n - NS * wb
    zb = next(z for z in range(min(CH, wb), 0, -1)
              if wb % z == 0 and z % 8 == 0)
    assert rem % 8 == 0 and rem <= wb

    def body(x_hbm, edge_hbm, sum_hbm, deg_hbm, *refs):
        src_v = refs[0:NBUF]
        dst_v = refs[NBUF:2 * NBUF]
        rows_v = refs[2 * NBUF:3 * NBUF]
        src_t, dst_t, rows_t, ones_v, zdeg_v, acc_sh, accd_sh = \
            refs[3 * NBUF:3 * NBUF + 7]
        semi = refs[3 * NBUF + 7:3 * NBUF + 7 + NBUF]
        semg = refs[3 * NBUF + 7 + NBUF:3 * NBUF + 7 + 2 * NBUF]
        sems = refs[3 * NBUF + 7 + 2 * NBUF:3 * NBUF + 7 + 3 * NBUF]
        semd = refs[3 * NBUF + 7 + 3 * NBUF:3 * NBUF + 7 + 4 * NBUF]
        semt = refs[3 * NBUF + 7 + 4 * NBUF]

        c = lax.axis_index("c")
        s = lax.axis_index("s")
        zero16 = jnp.zeros((LANES,), jnp.float32)
        zero32b = jnp.zeros((2 * LANES,), jnp.bfloat16)
        e0 = jnp.where(lax.iota(jnp.int32, LANES) == 0,
                       jnp.float32(1.0), jnp.float32(0.0))
        row_off = c * n

        def idx_issue(b, base):
            pltpu.async_copy(edge_hbm.at[pl.ds(base, CH)], src_v[b], semi[b])
            pltpu.async_copy(edge_hbm.at[pl.ds(e + base, CH)], dst_v[b],
                             semi[b])

        def idx_wait(b):
            pltpu.make_async_copy(edge_hbm.at[pl.ds(0, CH)], src_v[b],
                                  semi[b]).wait()
            pltpu.make_async_copy(edge_hbm.at[pl.ds(0, CH)], dst_v[b],
                                  semi[b]).wait()

        def gather_issue(b):
            for j in range(CH // LANES):
                sl = pl.ds(j * LANES, LANES)
                src_v[b][sl] = src_v[b][sl] + row_off
            pltpu.async_copy(x_hbm.at[src_v[b]], rows_v[b], semg[b])

        def gather_wait(b):
            pltpu.make_async_copy(x_hbm.at[src_v[b]], rows_v[b],
                                  semg[b]).wait()

        def scat_issue(b):
            pltpu.async_copy(rows_v[b], acc_sh.at[dst_v[b]], sems[b],
                             add=True)
            @pl.when(c == b % 2)
            def _():
                pltpu.async_copy(ones_v, accd_sh.at[dst_v[b]], semd[b],
                                 add=True)

        def scat_wait(b):
            pltpu.make_async_copy(rows_v[b], acc_sh.at[dst_v[b]],
                                  sems[b]).wait()

            @pl.when(c == b % 2)
            def _():
                pltpu.make_async_copy(ones_v, accd_sh.at[dst_v[b]],
                                      semd[b]).wait()

        def zrow(i, carry):
            for j in range(dh // (2 * LANES)):
                rows_v[0][i, pl.ds(j * 2 * LANES, 2 * LANES)] = zero32b
            ones_v[i, :] = e0
            return carry
        lax.fori_loop(0, CH, zrow, 0)

        def zdeg(i, carry):
            zdeg_v[i, :] = zero16
            return carry
        lax.fori_loop(0, wb, zdeg, 0)

        base_row = s * wb
        for j in range(wb // zb):
            pltpu.sync_copy(rows_v[0].at[pl.ds(0, zb)],
                            acc_sh.at[pl.ds(base_row + j * zb, zb)])
        pltpu.sync_copy(zdeg_v, accd_sh.at[pl.ds(base_row, wb)])
        if rem:
            @pl.when(s == 0)
            def _():
                pltpu.sync_copy(rows_v[0].at[pl.ds(0, rem)],
                                acc_sh.at[pl.ds(NS * wb, rem)])
                pltpu.sync_copy(zdeg_v.at[pl.ds(0, rem)],
                                accd_sh.at[pl.ds(NS * wb, rem)])
        plsc.subcore_barrier()

        ebase = s * epw

        pltpu.sync_copy(edge_hbm.at[pl.ds(ebase, CH)], src_v[0])
        pltpu.sync_copy(edge_hbm.at[pl.ds(e + ebase, CH)], dst_v[0])
        gather_issue(0)
        idx_issue(1, ebase + CH)
        idx_wait(1)
        gather_issue(1)
        idx_issue(2, ebase + 2 * CH)

        def sext(g, carry):
            for b in range(NBUF):
                i = g * NBUF + b
                b2 = (b + 2) % NBUF
                b3 = (b + 3) % NBUF

                @pl.when(i >= 3)
                def _():
                    scat_wait(b3)

                @pl.when(i + 3 < nch)
                def _():
                    idx_issue(b3, ebase + (i + 3) * CH)

                @pl.when(i + 2 < nch)
                def _():
                    idx_wait(b2)
                    gather_issue(b2)

                gather_wait(b)
                scat_issue(b)
            return carry
        lax.fori_loop(0, nch // NBUF, sext, 0)
        for k in (nch - 3, nch - 2, nch - 1):
            scat_wait(k % NBUF)

        if tail:
            bt = ebase + nch * CH
            pltpu.sync_copy(edge_hbm.at[pl.ds(bt, tail)], src_t)
            pltpu.sync_copy(edge_hbm.at[pl.ds(e + bt, tail)], dst_t)
            for j in range(tail // LANES):
                sl = pl.ds(j * LANES, LANES)
                src_t[sl] = src_t[sl] + row_off
            pltpu.async_copy(x_hbm.at[src_t], rows_t, semt).wait()
            pltpu.sync_copy(rows_t, acc_sh.at[dst_t], add=True)

            @pl.when(c == 1)
            def _():
                pltpu.sync_copy(ones_v.at[pl.ds(0, tail)],
                                accd_sh.at[dst_t], add=True)
        plsc.subcore_barrier()

        out_base = c * n + base_row
        pltpu.sync_copy(acc_sh.at[pl.ds(base_row, wb)],
                        sum_hbm.at[pl.ds(out_base, wb)])
        pltpu.sync_copy(accd_sh.at[pl.ds(base_row, wb)],
                        deg_hbm.at[pl.ds(out_base, wb)])
        if rem:
            @pl.when(s == 0)
            def _():
                pltpu.sync_copy(acc_sh.at[pl.ds(NS * wb, rem)],
                                sum_hbm.at[pl.ds(c * n + NS * wb, rem)])
                pltpu.sync_copy(accd_sh.at[pl.ds(NS * wb, rem)],
                                deg_hbm.at[pl.ds(c * n + NS * wb, rem)])

    scratch = (
        [pltpu.VMEM((CH,), jnp.int32) for _ in range(NBUF)]
        + [pltpu.VMEM((CH,), jnp.int32) for _ in range(NBUF)]
        + [pltpu.VMEM((CH, dh), jnp.bfloat16) for _ in range(NBUF)]
        + [
            pltpu.VMEM((max(tail, 8),), jnp.int32),
            pltpu.VMEM((max(tail, 8),), jnp.int32),
            pltpu.VMEM((max(tail, 8), dh), jnp.bfloat16),
            pltpu.VMEM((CH, LANES), jnp.float32),
            pltpu.VMEM((wb, LANES), jnp.float32),
            pltpu.MemorySpace.VMEM_SHARED((n, dh), jnp.bfloat16),
            pltpu.MemorySpace.VMEM_SHARED((n, LANES), jnp.float32),
        ]
        + [pltpu.SemaphoreType.DMA for _ in range(4 * NBUF + 1)]
    )

    return pl.kernel(
        body,
        out_type=(jax.ShapeDtypeStruct((NC * n, dh), jnp.bfloat16),
                  jax.ShapeDtypeStruct((NC * n, LANES), jnp.float32)),
        mesh=plsc.VectorSubcoreMesh(core_axis_name="c", subcore_axis_name="s"),
        compiler_params=pltpu.CompilerParams(use_tc_tiling_on_sc=False),
        scratch_types=scratch,
    )


def _tc_self_body(x_ref, w_ref, b_ref, o_ref):
    o_ref[...] = (
        jnp.dot(x_ref[...], w_ref[0], preferred_element_type=jnp.float32)
        + b_ref[...])


def _tc_body(o1_ref, s_ref, d_ref, w_ref, o_ref):
    ssum = jnp.concatenate([s_ref[0], s_ref[1]],
                           axis=1).astype(jnp.float32)
    deg = jnp.sum(d_ref[0] + d_ref[1], axis=1, keepdims=True)
    h = ssum / jnp.maximum(deg, 1.0)
    o_ref[...] = (
        o1_ref[...]
        + jnp.dot(h, w_ref[1], preferred_element_type=jnp.float32))


def kernel(x, edge_index, W_neigh, b_neigh):
    n, d = x.shape
    e = edge_index.shape[1]
    dout = W_neigh.shape[1]
    dh = d // NC

    xs = x.reshape(n, NC, dh).swapaxes(0, 1).reshape(NC * n, dh)
    xs = xs.astype(jnp.bfloat16)
    sums, degs = _sc_scatter_kernel(n, d, e)(xs, edge_index.reshape(-1))
    sums = sums.reshape(NC, n, dh)
    degs = degs.reshape(NC, n, LANES)
    w = W_neigh.reshape(2, d, dout)
    b = b_neigh.reshape(1, dout)

    rb = 2000 if n % 2000 == 0 else n
    out1 = pl.pallas_call(
        _tc_self_body,
        grid=(n // rb,),
        in_specs=[
            pl.BlockSpec((rb, d), lambda i: (i, 0)),
            pl.BlockSpec((2, d, dout), lambda i: (0, 0, 0)),
            pl.BlockSpec((1, dout), lambda i: (0, 0)),
        ],
        out_specs=pl.BlockSpec((rb, dout), lambda i: (i, 0)),
        out_shape=jax.ShapeDtypeStruct((n, dout), jnp.float32),
    )(x, w, b)
    out = pl.pallas_call(
        _tc_body,
        grid=(n // rb,),
        in_specs=[
            pl.BlockSpec((rb, dout), lambda i: (i, 0)),
            pl.BlockSpec((NC, rb, dh), lambda i: (0, i, 0)),
            pl.BlockSpec((NC, rb, LANES), lambda i: (0, i, 0)),
            pl.BlockSpec((2, d, dout), lambda i: (0, 0, 0)),
        ],
        out_specs=pl.BlockSpec((rb, dout), lambda i: (i, 0)),
        out_shape=jax.ShapeDtypeStruct((n, dout), jnp.float32),
    )(out1, sums, degs, w)
    return out
